# baseline (speedup 1.0000x reference)
import jax
import jax.numpy as jnp
from jax import lax
from jax.experimental import pallas as pl
from jax.experimental.pallas import tpu as pltpu
from jax.experimental.pallas import tpu_sc as plsc


def _gather_rows_sc(x_hbm, o0, o1, o2):
    c = lax.axis_index("c")
    s = lax.axis_index("s")

    @pl.when(jnp.logical_and(c == 0, s == 0))
    def _():
        pltpu.sync_copy(x_hbm.at[0], o0)
        pltpu.sync_copy(x_hbm.at[1], o1)
        pltpu.sync_copy(x_hbm.at[2], o2)


def kernel(x):
    row = jax.ShapeDtypeStruct((128,), jnp.float32)
    out = pl.kernel(
        _gather_rows_sc,
        out_type=(row, row, row),
        mesh=plsc.VectorSubcoreMesh(core_axis_name="c", subcore_axis_name="s"),
    )(x)
    return tuple(out)
